# baseline (speedup 1.0000x reference)
import functools

import jax
import jax.numpy as jnp
from jax import lax
from jax.experimental import pallas as pl
from jax.experimental.pallas import tpu as pltpu
from jax.experimental.pallas import tpu_sc as plsc

NUM_BUCKETS = 100000
HASH_DIM = 112
MODEL_DIM = 2048

_NC = 2
_NS = 16
_NW = _NC * _NS
_LANES = 16

_CHUNK = 128


def _gather_sum_sc(ids_t, ids_t1, ids_t2, emb):
    n = ids_t.shape[0]
    assert n % (_NW * _CHUNK) == 0
    per_w = n // _NW
    nchunks = per_w // _CHUNK
    mesh = plsc.VectorSubcoreMesh(core_axis_name="c", subcore_axis_name="s")

    @functools.partial(
        pl.kernel,
        out_type=jax.ShapeDtypeStruct((n, HASH_DIM), jnp.float32),
        mesh=mesh,
        scratch_types=[
            pltpu.VMEM((_CHUNK,), jnp.int32),
            pltpu.VMEM((_CHUNK,), jnp.int32),
            pltpu.VMEM((_CHUNK,), jnp.int32),
            pltpu.VMEM((_CHUNK,), jnp.int32),
            pltpu.VMEM((_CHUNK,), jnp.int32),
            pltpu.VMEM((_CHUNK,), jnp.int32),
            pltpu.VMEM((_CHUNK,), jnp.int32),
            pltpu.VMEM((_CHUNK, HASH_DIM), jnp.float32),
            pltpu.VMEM((_CHUNK, HASH_DIM), jnp.float32),
            pltpu.VMEM((_CHUNK, HASH_DIM), jnp.float32),
            pltpu.VMEM((_CHUNK, HASH_DIM), jnp.float32),
            pltpu.SemaphoreType.DMA,
        ],
        compiler_params=pltpu.CompilerParams(use_tc_tiling_on_sc=False),
    )
    def k(t_hbm, t1_hbm, t2_hbm, emb_hbm, out_hbm,
          t_v, t1_v, t2_v, i0, i1, i2, i3, b0, b1, b2, b3, sem):
        wid = lax.axis_index("s") * _NC + lax.axis_index("c")

        def chunk_body(c, carry):
            base = wid * per_w + c * _CHUNK
            pltpu.sync_copy(t_hbm.at[pl.ds(base, _CHUNK)], t_v)
            pltpu.sync_copy(t1_hbm.at[pl.ds(base, _CHUNK)], t1_v)
            pltpu.sync_copy(t2_hbm.at[pl.ds(base, _CHUNK)], t2_v)

            def hash_body(g, carry):
                s = pl.ds(g * _LANES, _LANES)
                a = t_v[s]
                b = t1_v[s]
                d = t2_v[s]
                i0[s] = lax.rem(b ^ (a * 62), NUM_BUCKETS)
                i1[s] = lax.rem(b ^ (a * 194), NUM_BUCKETS)
                i2[s] = lax.rem(d ^ (b * 34) ^ (a * 51), NUM_BUCKETS)
                i3[s] = lax.rem(d ^ (b * 106) ^ (a * 159), NUM_BUCKETS)
                return carry

            lax.fori_loop(0, _CHUNK // _LANES, hash_body, 0)

            cp0 = pltpu.async_copy(emb_hbm.at[i0], b0, sem)
            cp1 = pltpu.async_copy(emb_hbm.at[i1], b1, sem)
            cp2 = pltpu.async_copy(emb_hbm.at[i2], b2, sem)
            cp3 = pltpu.async_copy(emb_hbm.at[i3], b3, sem)
            cp0.wait()
            cp1.wait()
            cp2.wait()
            cp3.wait()

            def add_body(r, carry):
                for j in range(HASH_DIM // _LANES):
                    s = pl.ds(j * _LANES, _LANES)
                    b0[r, s] = b0[r, s] + b1[r, s] + b2[r, s] + b3[r, s]
                return carry

            lax.fori_loop(0, _CHUNK, add_body, 0)

            pltpu.sync_copy(b0, out_hbm.at[pl.ds(base, _CHUNK)])
            return carry

        lax.fori_loop(0, nchunks, chunk_body, 0)

    return k(ids_t, ids_t1, ids_t2, emb)


def _proj_tc(total, proj_wt):
    n = total.shape[0]
    bm = 512
    assert n % bm == 0

    def body(x_ref, w_ref, o_ref):
        x = x_ref[...] * 0.25
        o_ref[...] = lax.dot_general(
            x, w_ref[...], (((1,), (0,)), ((), ())),
            preferred_element_type=jnp.float32)

    return pl.pallas_call(
        body,
        grid=(n // bm,),
        in_specs=[
            pl.BlockSpec((bm, HASH_DIM), lambda i: (i, 0)),
            pl.BlockSpec((HASH_DIM, MODEL_DIM), lambda i: (0, 0)),
        ],
        out_specs=pl.BlockSpec((bm, MODEL_DIM), lambda i: (i, 0)),
        out_shape=jax.ShapeDtypeStruct((n, MODEL_DIM), jnp.float32),
    )(total, proj_wt)


def kernel(input_ids, emb, proj_w):
    bsz, seqlen = input_ids.shape
    z = jnp.zeros((bsz, 1), input_ids.dtype)
    ids_t = input_ids.reshape(-1)
    ids_t1 = jnp.concatenate([z, input_ids[:, :-1]], axis=1).reshape(-1)
    ids_t2 = jnp.concatenate([z, z, input_ids[:, :-2]], axis=1).reshape(-1)
    total = _gather_sum_sc(ids_t, ids_t1, ids_t2, emb)
    out = _proj_tc(total, proj_w.T)
    return out.reshape(bsz, seqlen, MODEL_DIM)
